# baseline (speedup 1.0000x reference)
import jax
import jax.numpy as jnp
from jax.experimental import pallas as pl


_GRID = 32


def _body(idx_ref, emb_ref, out_ref):
    i = pl.program_id(0)
    rs = jnp.sum(emb_ref[...], axis=0, keepdims=True)
    idx = idx_ref[...]
    table = jnp.broadcast_to(rs, (idx.shape[0], 128))
    vals = jnp.take_along_axis(table, idx, axis=1)
    part = jnp.sum(vals, keepdims=True).reshape(1, 1)

    @pl.when(i == 0)
    def _():
        out_ref[...] = part - 1.0

    @pl.when(i > 0)
    def _():
        out_ref[...] += part


def kernel(input_batch, emb36a, emb36b, emb48):
    del emb36a, emb36b
    n, c = input_batch.shape
    block = n // _GRID
    emb_t = jnp.zeros((emb48.shape[1], 128), jnp.float32).at[:, : emb48.shape[0]].set(emb48.T)
    out = pl.pallas_call(
        _body,
        grid=(_GRID,),
        in_specs=[
            pl.BlockSpec((block, c), lambda i: (i, 0)),
            pl.BlockSpec(emb_t.shape, lambda i: (0, 0)),
        ],
        out_specs=pl.BlockSpec((1, 1), lambda i: (0, 0)),
        out_shape=jax.ShapeDtypeStruct((1, 1), jnp.float32),
    )(input_batch.astype(jnp.int32), emb_t)
    return out[0, 0]


# speedup vs baseline: 1.3635x; 1.3635x over previous
import jax
import jax.numpy as jnp
from jax.experimental import pallas as pl


_GRID = 8


def _body(idx_ref, emb_ref, out_ref):
    i = pl.program_id(0)
    rs = jnp.sum(emb_ref[...], axis=0, keepdims=True)
    idx = idx_ref[...]
    table = jnp.broadcast_to(rs, (idx.shape[0], 128))
    vals = jnp.take_along_axis(table, idx, axis=1)
    part = jnp.sum(vals, keepdims=True).reshape(1, 1)

    @pl.when(i == 0)
    def _():
        out_ref[...] = part - 1.0

    @pl.when(i > 0)
    def _():
        out_ref[...] += part


def kernel(input_batch, emb36a, emb36b, emb48):
    del emb36a, emb36b
    n, c = input_batch.shape
    block = n // _GRID
    emb_t = jnp.zeros((emb48.shape[1], 128), jnp.float32).at[:, : emb48.shape[0]].set(emb48.T)
    out = pl.pallas_call(
        _body,
        grid=(_GRID,),
        in_specs=[
            pl.BlockSpec((block, c), lambda i: (i, 0)),
            pl.BlockSpec(emb_t.shape, lambda i: (0, 0)),
        ],
        out_specs=pl.BlockSpec((1, 1), lambda i: (0, 0)),
        out_shape=jax.ShapeDtypeStruct((1, 1), jnp.float32),
    )(input_batch.astype(jnp.int32), emb_t)
    return out[0, 0]


# speedup vs baseline: 1.3746x; 1.0082x over previous
import jax
import jax.numpy as jnp
from jax.experimental import pallas as pl


_GRID = 4


def _body(idx_ref, emb_ref, out_ref):
    i = pl.program_id(0)
    rs = jnp.sum(emb_ref[...], axis=0, keepdims=True)
    idx = idx_ref[...]
    table = jnp.broadcast_to(rs, (idx.shape[0], 128))
    vals = jnp.take_along_axis(table, idx, axis=1)
    part = jnp.sum(vals, keepdims=True).reshape(1, 1)

    @pl.when(i == 0)
    def _():
        out_ref[...] = part - 1.0

    @pl.when(i > 0)
    def _():
        out_ref[...] += part


def kernel(input_batch, emb36a, emb36b, emb48):
    del emb36a, emb36b
    n, c = input_batch.shape
    block = n // _GRID
    emb_t = jnp.zeros((emb48.shape[1], 128), jnp.float32).at[:, : emb48.shape[0]].set(emb48.T)
    out = pl.pallas_call(
        _body,
        grid=(_GRID,),
        in_specs=[
            pl.BlockSpec((block, c), lambda i: (i, 0)),
            pl.BlockSpec(emb_t.shape, lambda i: (0, 0)),
        ],
        out_specs=pl.BlockSpec((1, 1), lambda i: (0, 0)),
        out_shape=jax.ShapeDtypeStruct((1, 1), jnp.float32),
    )(input_batch.astype(jnp.int32), emb_t)
    return out[0, 0]
